# baseline (speedup 1.0000x reference)
import jax
import jax.numpy as jnp
from jax.experimental import pallas as pl
from jax.experimental.pallas import tpu as pltpu

N, D = 16384, 64
CHUNKS = (4096, 4096, 4096, 2048, 1024, 512, 512)
OFFS = tuple(sum(CHUNKS[:k]) for k in range(len(CHUNKS)))
NB = len(CHUNKS)
assert sum(CHUNKS) == N


def _body(u_hbm, i_hbm, o_hbm, *rest):
    ubufs = rest[0:NB]
    ibufs = rest[NB:2 * NB]
    o_v = rest[2 * NB]
    sems = rest[2 * NB + 1:2 * NB + 1 + NB]
    osem = rest[2 * NB + 1 + NB]

    def start(k):
        cu = pltpu.make_async_copy(
            u_hbm.at[:, pl.ds(OFFS[k], CHUNKS[k])], ubufs[k], sems[k])
        ci = pltpu.make_async_copy(
            i_hbm.at[:, pl.ds(OFFS[k], CHUNKS[k])], ibufs[k], sems[k])
        cu.start()
        ci.start()
        return cu, ci

    pend = [start(k) for k in range(NB)]
    for k in range(NB):
        cu, ci = pend[k]
        cu.wait()
        ci.wait()
        o_v[pl.ds(OFFS[k], CHUNKS[k])] = jnp.sum(
            ubufs[k][...] * ibufs[k][...], axis=0)
    out_cp = pltpu.make_async_copy(o_v, o_hbm, osem)
    out_cp.start()
    out_cp.wait()


def kernel(gu, gi):
    return pl.pallas_call(
        _body,
        in_specs=[
            pl.BlockSpec(memory_space=pltpu.HBM),
            pl.BlockSpec(memory_space=pltpu.HBM),
        ],
        out_specs=pl.BlockSpec(memory_space=pltpu.HBM),
        out_shape=jax.ShapeDtypeStruct((N,), jnp.float32),
        scratch_shapes=(
            [pltpu.VMEM((D, c), jnp.float32) for c in CHUNKS]
            + [pltpu.VMEM((D, c), jnp.float32) for c in CHUNKS]
            + [pltpu.VMEM((N,), jnp.float32)]
            + [pltpu.SemaphoreType.DMA for _ in range(NB + 1)]
        ),
    )(pltpu.with_memory_space_constraint(gu.T, pltpu.HBM),
      pltpu.with_memory_space_constraint(gi.T, pltpu.HBM))


# speedup vs baseline: 1.0051x; 1.0051x over previous
import jax
import jax.numpy as jnp
from jax.experimental import pallas as pl
from jax.experimental.pallas import tpu as pltpu

N, D = 16384, 64
BC = 4096
NB = N // BC


def _body(u_hbm, i_hbm, o_hbm, *rest):
    ubufs = rest[0:NB]
    ibufs = rest[NB:2 * NB]
    o_v = rest[2 * NB]
    sems = rest[2 * NB + 1:2 * NB + 1 + NB]
    osem = rest[2 * NB + 1 + NB]

    def start(k):
        cu = pltpu.make_async_copy(
            u_hbm.at[:, pl.ds(k * BC, BC)], ubufs[k], sems[k])
        ci = pltpu.make_async_copy(
            i_hbm.at[:, pl.ds(k * BC, BC)], ibufs[k], sems[k])
        cu.start()
        ci.start()
        return cu, ci

    pend = [start(k) for k in range(NB)]
    for k in range(NB):
        cu, ci = pend[k]
        cu.wait()
        ci.wait()
        o_v[pl.ds(k * BC, BC)] = jnp.sum(
            ubufs[k][...] * ibufs[k][...], axis=0)
    out_cp = pltpu.make_async_copy(o_v, o_hbm, osem)
    out_cp.start()
    out_cp.wait()


def kernel(gu, gi):
    return pl.pallas_call(
        _body,
        in_specs=[
            pl.BlockSpec(memory_space=pltpu.HBM),
            pl.BlockSpec(memory_space=pltpu.HBM),
        ],
        out_specs=pl.BlockSpec(memory_space=pltpu.HBM),
        out_shape=jax.ShapeDtypeStruct((N,), jnp.float32),
        scratch_shapes=(
            [pltpu.VMEM((D, BC), jnp.float32) for _ in range(2 * NB)]
            + [pltpu.VMEM((N,), jnp.float32)]
            + [pltpu.SemaphoreType.DMA for _ in range(NB + 1)]
        ),
    )(pltpu.with_memory_space_constraint(gu.T, pltpu.HBM),
      pltpu.with_memory_space_constraint(gi.T, pltpu.HBM))
